# baseline (speedup 1.0000x reference)
import functools

import jax
import jax.numpy as jnp
from jax import lax
from jax.experimental import pallas as pl
from jax.experimental.pallas import tpu as pltpu
from jax.experimental.pallas import tpu_sc as plsc

N_NODES = 10000
N_EDGES = 320000
NC = 2
NS = 16
NW = NC * NS
E_PER_W = N_EDGES // NW
ROWS_PER_TILE = N_NODES // NS


def _make_sc_aggregate(d: int, chunk: int, n_chunks: int, acc_rows: int):
  mesh = plsc.VectorSubcoreMesh(core_axis_name="c", subcore_axis_name="s")
  zrows = acc_rows // NS

  @functools.partial(
      pl.kernel,
      out_type=jax.ShapeDtypeStruct((NC * N_NODES, d), jnp.float32),
      mesh=mesh,
      scratch_types=dict(
          pk_v=pltpu.VMEM((n_chunks, chunk), jnp.int32),
          src_a=pltpu.VMEM((chunk,), jnp.int32),
          dst_a=pltpu.VMEM((chunk,), jnp.int32),
          src_b=pltpu.VMEM((chunk,), jnp.int32),
          dst_b=pltpu.VMEM((chunk,), jnp.int32),
          buf_a=pltpu.VMEM((chunk, d), jnp.float32),
          buf_b=pltpu.VMEM((chunk, d), jnp.float32),
          acc_sh=pltpu.VMEM_SHARED((acc_rows, d), jnp.float32),
          sem_a=pltpu.SemaphoreType.DMA,
          sem_b=pltpu.SemaphoreType.DMA,
      ),
      compiler_params=pltpu.CompilerParams(use_tc_tiling_on_sc=False),
  )
  def agg(table_hbm, pk_hbm, zeros_hbm, out_hbm,
          pk_v, src_a, dst_a, src_b, dst_b, buf_a, buf_b, acc_sh,
          sem_a, sem_b):
    c = lax.axis_index("c")
    s = lax.axis_index("s")
    wid = c * NS + s

    pltpu.sync_copy(pk_hbm.at[wid], pk_v)

    def unpack(i, src_x, dst_x):
      for g in range(chunk // 16):
        pk = pk_v[i, pl.ds(g * 16, 16)]
        src_x[pl.ds(g * 16, 16)] = lax.shift_right_logical(pk, 16)
        dst_x[pl.ds(g * 16, 16)] = lax.bitwise_and(pk, 0xFFFF)

    def gather(src_x, buf, sem):
      pltpu.async_copy(table_hbm.at[src_x], buf, sem)

    def wait(src_x, buf, sem):
      pltpu.make_async_copy(table_hbm.at[src_x], buf, sem).wait()

    def scatter(dst_x, buf):
      pltpu.sync_copy(buf, acc_sh.at[dst_x], add=True)

    unpack(0, src_a, dst_a)
    gather(src_a, buf_a, sem_a)

    pltpu.sync_copy(zeros_hbm, buf_b)
    z0 = s * zrows
    zn_full, zrem = divmod(zrows, chunk)
    for k in range(zn_full):
      pltpu.sync_copy(buf_b, acc_sh.at[pl.ds(z0 + k * chunk, chunk)])
    if zrem:
      pltpu.sync_copy(buf_b.at[pl.ds(0, zrem)],
                      acc_sh.at[pl.ds(z0 + zn_full * chunk, zrem)])
    plsc.subcore_barrier()

    def body(j, carry):
      i0 = 2 * j
      unpack(i0 + 1, src_b, dst_b)
      gather(src_b, buf_b, sem_b)
      wait(src_a, buf_a, sem_a)
      scatter(dst_a, buf_a)
      unpack(i0 + 2, src_a, dst_a)
      gather(src_a, buf_a, sem_a)
      wait(src_b, buf_b, sem_b)
      scatter(dst_b, buf_b)
      return carry

    lax.fori_loop(0, n_chunks // 2, body, 0)
    wait(src_a, buf_a, sem_a)
    scatter(dst_a, buf_a)
    plsc.subcore_barrier()

    row0 = s * ROWS_PER_TILE
    out0 = c * N_NODES + row0
    n_full, rem = divmod(ROWS_PER_TILE, chunk)
    for k in range(n_full):
      pltpu.sync_copy(acc_sh.at[pl.ds(row0 + k * chunk, chunk)], buf_a)
      pltpu.sync_copy(buf_a, out_hbm.at[pl.ds(out0 + k * chunk, chunk)])
    if rem:
      pltpu.sync_copy(acc_sh.at[pl.ds(row0 + n_full * chunk, rem)],
                      buf_b.at[pl.ds(0, rem)])
      pltpu.sync_copy(buf_b.at[pl.ds(0, rem)],
                      out_hbm.at[pl.ds(out0 + n_full * chunk, rem)])

  return agg


_D1 = 144
_D2 = 48
_BLK = 1000
_GRID = N_NODES // _BLK


def _dense1_body(x_ref, a0_ref, a1_ref, ws1_ref, wn1_ref, b1_ref,
                 ws2_ref, wn2_ref, b2_ref, g2p_ref, s2b_ref, dinv_ref):
  a = a0_ref[...] + a1_ref[...]
  dinv = 1.0 / jnp.maximum(a[:, 128:129], 1.0)
  mean = a[:, :128] * dinv
  h1 = x_ref[...] @ ws1_ref[...] + mean @ wn1_ref[...] + b1_ref[...]
  h1 = jnp.maximum(h1, 0.0)
  g2 = h1 @ wn2_ref[...]
  g2p_ref[...] = jnp.concatenate(
      [g2, jnp.zeros((g2.shape[0], _D2 - g2.shape[1]), jnp.float32)], axis=1)
  s2b_ref[...] = h1 @ ws2_ref[...] + b2_ref[...]
  dinv_ref[...] = jnp.broadcast_to(dinv, (dinv.shape[0], 8))


def _dense2_body(a0_ref, a1_ref, s2b_ref, dinv_ref, out_ref):
  a = a0_ref[...] + a1_ref[...]
  out_ref[...] = s2b_ref[...] + a[:, :40] * dinv_ref[:, :1]


_C1, _NCH1 = 80, 125
_C2, _NCH2 = 96, 105
_ACC2_ROWS = N_NODES + 16


def kernel(x, edge_index, W_self1, W_neigh1, b1, W_self2, W_neigh2, b2):
  src = edge_index[0].astype(jnp.int32)
  dst = edge_index[1].astype(jnp.int32)
  pk = ((src << 16) | dst).reshape(NW, E_PER_W)
  pk1 = pk.reshape(NW, _NCH1, _C1)
  pad_dst = N_NODES + (jnp.arange(NW, dtype=jnp.int32) % NS)
  pad_blk = jnp.broadcast_to(pad_dst[:, None], (NW, _NCH2 * _C2 - E_PER_W))
  pk2 = jnp.concatenate([pk, pad_blk], axis=1).reshape(NW, _NCH2, _C2)
  n_classes = W_self2.shape[1]

  xp = jnp.concatenate(
      [x, jnp.ones((N_NODES, 1), jnp.float32),
       jnp.zeros((N_NODES, _D1 - x.shape[1] - 1), jnp.float32)], axis=1)
  z1 = jnp.zeros((_C1, _D1), jnp.float32)
  acc1 = _make_sc_aggregate(_D1, _C1, _NCH1, N_NODES)(xp, pk1, z1)

  g2p, s2b, dinv = pl.pallas_call(
      _dense1_body,
      grid=(_GRID,),
      in_specs=[
          pl.BlockSpec((_BLK, 128), lambda i: (i, 0)),
          pl.BlockSpec((_BLK, _D1), lambda i: (i, 0)),
          pl.BlockSpec((_BLK, _D1), lambda i: (i + _GRID, 0)),
          pl.BlockSpec((128, 128), lambda i: (0, 0)),
          pl.BlockSpec((128, 128), lambda i: (0, 0)),
          pl.BlockSpec((1, 128), lambda i: (0, 0)),
          pl.BlockSpec((128, n_classes), lambda i: (0, 0)),
          pl.BlockSpec((128, n_classes), lambda i: (0, 0)),
          pl.BlockSpec((1, n_classes), lambda i: (0, 0)),
      ],
      out_specs=[
          pl.BlockSpec((_BLK, _D2), lambda i: (i, 0)),
          pl.BlockSpec((_BLK, n_classes), lambda i: (i, 0)),
          pl.BlockSpec((_BLK, 8), lambda i: (i, 0)),
      ],
      out_shape=[
          jax.ShapeDtypeStruct((N_NODES, _D2), jnp.float32),
          jax.ShapeDtypeStruct((N_NODES, n_classes), jnp.float32),
          jax.ShapeDtypeStruct((N_NODES, 8), jnp.float32),
      ],
  )(x, acc1, acc1, W_self1, W_neigh1, b1.reshape(1, -1),
    W_self2, W_neigh2, b2.reshape(1, -1))

  z2 = jnp.zeros((_C2, _D2), jnp.float32)
  acc2 = _make_sc_aggregate(_D2, _C2, _NCH2, _ACC2_ROWS)(g2p, pk2, z2)

  out = pl.pallas_call(
      _dense2_body,
      grid=(_GRID,),
      in_specs=[
          pl.BlockSpec((_BLK, _D2), lambda i: (i, 0)),
          pl.BlockSpec((_BLK, _D2), lambda i: (i + _GRID, 0)),
          pl.BlockSpec((_BLK, n_classes), lambda i: (i, 0)),
          pl.BlockSpec((_BLK, 8), lambda i: (i, 0)),
      ],
      out_specs=pl.BlockSpec((_BLK, n_classes), lambda i: (i, 0)),
      out_shape=jax.ShapeDtypeStruct((N_NODES, n_classes), jnp.float32),
  )(acc2, acc2, s2b, dinv)
  return out


# speedup vs baseline: 1.0793x; 1.0793x over previous
import functools

import jax
import jax.numpy as jnp
from jax import lax
from jax.experimental import pallas as pl
from jax.experimental.pallas import tpu as pltpu
from jax.experimental.pallas import tpu_sc as plsc

N_NODES = 10000
N_EDGES = 320000
NC = 2
NS = 16
NW = NC * NS
E_PER_W = N_EDGES // NW
ROWS_PER_TILE = N_NODES // NS


def _make_sc_aggregate(d: int, chunk: int, n_chunks: int, acc_rows: int):
  mesh = plsc.VectorSubcoreMesh(core_axis_name="c", subcore_axis_name="s")
  zrows = acc_rows // NS
  half = (n_chunks + 1) // 2

  @functools.partial(
      pl.kernel,
      out_type=jax.ShapeDtypeStruct((NC * N_NODES, d), jnp.float32),
      mesh=mesh,
      scratch_types=dict(
          src_h=pltpu.VMEM((half, chunk), jnp.int32),
          dst_h=pltpu.VMEM((half, chunk), jnp.int32),
          buf_a=pltpu.VMEM((chunk, d), jnp.float32),
          buf_b=pltpu.VMEM((chunk, d), jnp.float32),
          acc_sh=pltpu.VMEM_SHARED((acc_rows, d), jnp.float32),
          sem_a=pltpu.SemaphoreType.DMA,
          sem_b=pltpu.SemaphoreType.DMA,
      ),
      compiler_params=pltpu.CompilerParams(use_tc_tiling_on_sc=False),
  )
  def agg(table_hbm, src_hbm, dst_hbm, zeros_hbm, out_hbm,
          src_h, dst_h, buf_a, buf_b, acc_sh, sem_a, sem_b):
    c = lax.axis_index("c")
    s = lax.axis_index("s")
    wid = c * NS + s

    def gather(l, buf, sem):
      pltpu.async_copy(table_hbm.at[src_h.at[l]], buf, sem)

    def wait(l, buf, sem):
      pltpu.make_async_copy(table_hbm.at[src_h.at[l]], buf, sem).wait()

    def scatter(l, buf):
      pltpu.sync_copy(buf, acc_sh.at[dst_h.at[l]], add=True)

    def stage_idx(base, n):
      pltpu.sync_copy(src_hbm.at[wid, pl.ds(base, n)], src_h.at[pl.ds(0, n)])
      pltpu.sync_copy(dst_hbm.at[wid, pl.ds(base, n)], dst_h.at[pl.ds(0, n)])

    def run_phase(n):
      gather(0, buf_a, sem_a)
      n_pairs = (n - 1) // 2

      def body(j, carry):
        l = 2 * j + 1
        gather(l, buf_b, sem_b)
        wait(l - 1, buf_a, sem_a)
        scatter(l - 1, buf_a)
        gather(l + 1, buf_a, sem_a)
        wait(l, buf_b, sem_b)
        scatter(l, buf_b)
        return carry

      lax.fori_loop(0, n_pairs, body, 0)
      if (n - 1) % 2 == 0:
        wait(n - 1, buf_a, sem_a)
        scatter(n - 1, buf_a)
      else:
        gather(n - 1, buf_b, sem_b)
        wait(n - 2, buf_a, sem_a)
        scatter(n - 2, buf_a)
        wait(n - 1, buf_b, sem_b)
        scatter(n - 1, buf_b)

    stage_idx(0, half)
    pltpu.sync_copy(zeros_hbm, buf_b)
    z0 = s * zrows
    zn_full, zrem = divmod(zrows, chunk)
    for k in range(zn_full):
      pltpu.sync_copy(buf_b, acc_sh.at[pl.ds(z0 + k * chunk, chunk)])
    if zrem:
      pltpu.sync_copy(buf_b.at[pl.ds(0, zrem)],
                      acc_sh.at[pl.ds(z0 + zn_full * chunk, zrem)])
    plsc.subcore_barrier()

    run_phase(half)
    stage_idx(half, n_chunks - half)
    run_phase(n_chunks - half)
    plsc.subcore_barrier()

    row0 = s * ROWS_PER_TILE
    out0 = c * N_NODES + row0
    n_full, rem = divmod(ROWS_PER_TILE, chunk)
    for k in range(n_full):
      pltpu.sync_copy(acc_sh.at[pl.ds(row0 + k * chunk, chunk)], buf_a)
      pltpu.sync_copy(buf_a, out_hbm.at[pl.ds(out0 + k * chunk, chunk)])
    if rem:
      pltpu.sync_copy(acc_sh.at[pl.ds(row0 + n_full * chunk, rem)],
                      buf_b.at[pl.ds(0, rem)])
      pltpu.sync_copy(buf_b.at[pl.ds(0, rem)],
                      out_hbm.at[pl.ds(out0 + n_full * chunk, rem)])

  return agg


_D1 = 144
_D2 = 48
_BLK = 1000
_GRID = N_NODES // _BLK


def _dense1_body(x_ref, a0_ref, a1_ref, ws1_ref, wn1_ref, b1_ref,
                 ws2_ref, wn2_ref, b2_ref, g2p_ref, s2b_ref, dinv_ref):
  a = a0_ref[...] + a1_ref[...]
  dinv = 1.0 / jnp.maximum(a[:, 128:129], 1.0)
  mean = a[:, :128] * dinv
  h1 = x_ref[...] @ ws1_ref[...] + mean @ wn1_ref[...] + b1_ref[...]
  h1 = jnp.maximum(h1, 0.0)
  g2 = h1 @ wn2_ref[...]
  g2p_ref[...] = jnp.concatenate(
      [g2, jnp.zeros((g2.shape[0], _D2 - g2.shape[1]), jnp.float32)], axis=1)
  s2b_ref[...] = h1 @ ws2_ref[...] + b2_ref[...]
  dinv_ref[...] = jnp.broadcast_to(dinv, (dinv.shape[0], 8))


def _dense2_body(a0_ref, a1_ref, s2b_ref, dinv_ref, out_ref):
  a = a0_ref[...] + a1_ref[...]
  out_ref[...] = s2b_ref[...] + a[:, :40] * dinv_ref[:, :1]


_C1, _NCH1 = 80, 125
_C2, _NCH2 = 80, 125


def kernel(x, edge_index, W_self1, W_neigh1, b1, W_self2, W_neigh2, b2):
  src3 = edge_index[0].astype(jnp.int32).reshape(NW, _NCH1, _C1)
  dst3 = edge_index[1].astype(jnp.int32).reshape(NW, _NCH1, _C1)
  n_classes = W_self2.shape[1]

  xp = jnp.concatenate(
      [x, jnp.ones((N_NODES, 1), jnp.float32),
       jnp.zeros((N_NODES, _D1 - x.shape[1] - 1), jnp.float32)], axis=1)
  z1 = jnp.zeros((_C1, _D1), jnp.float32)
  acc1 = _make_sc_aggregate(_D1, _C1, _NCH1, N_NODES)(xp, src3, dst3, z1)

  g2p, s2b, dinv = pl.pallas_call(
      _dense1_body,
      grid=(_GRID,),
      in_specs=[
          pl.BlockSpec((_BLK, 128), lambda i: (i, 0)),
          pl.BlockSpec((_BLK, _D1), lambda i: (i, 0)),
          pl.BlockSpec((_BLK, _D1), lambda i: (i + _GRID, 0)),
          pl.BlockSpec((128, 128), lambda i: (0, 0)),
          pl.BlockSpec((128, 128), lambda i: (0, 0)),
          pl.BlockSpec((1, 128), lambda i: (0, 0)),
          pl.BlockSpec((128, n_classes), lambda i: (0, 0)),
          pl.BlockSpec((128, n_classes), lambda i: (0, 0)),
          pl.BlockSpec((1, n_classes), lambda i: (0, 0)),
      ],
      out_specs=[
          pl.BlockSpec((_BLK, _D2), lambda i: (i, 0)),
          pl.BlockSpec((_BLK, n_classes), lambda i: (i, 0)),
          pl.BlockSpec((_BLK, 8), lambda i: (i, 0)),
      ],
      out_shape=[
          jax.ShapeDtypeStruct((N_NODES, _D2), jnp.float32),
          jax.ShapeDtypeStruct((N_NODES, n_classes), jnp.float32),
          jax.ShapeDtypeStruct((N_NODES, 8), jnp.float32),
      ],
  )(x, acc1, acc1, W_self1, W_neigh1, b1.reshape(1, -1),
    W_self2, W_neigh2, b2.reshape(1, -1))

  z2 = jnp.zeros((_C2, _D2), jnp.float32)
  acc2 = _make_sc_aggregate(_D2, _C2, _NCH2, N_NODES)(g2p, src3, dst3, z2)

  out = pl.pallas_call(
      _dense2_body,
      grid=(_GRID,),
      in_specs=[
          pl.BlockSpec((_BLK, _D2), lambda i: (i, 0)),
          pl.BlockSpec((_BLK, _D2), lambda i: (i + _GRID, 0)),
          pl.BlockSpec((_BLK, n_classes), lambda i: (i, 0)),
          pl.BlockSpec((_BLK, 8), lambda i: (i, 0)),
      ],
      out_specs=pl.BlockSpec((_BLK, n_classes), lambda i: (i, 0)),
      out_shape=jax.ShapeDtypeStruct((N_NODES, n_classes), jnp.float32),
  )(acc2, acc2, s2b, dinv)
  return out


# speedup vs baseline: 1.1151x; 1.0332x over previous
import functools

import jax
import jax.numpy as jnp
from jax import lax
from jax.experimental import pallas as pl
from jax.experimental.pallas import tpu as pltpu
from jax.experimental.pallas import tpu_sc as plsc

N_NODES = 10000
N_EDGES = 320000
NC = 2
NS = 16
NW = NC * NS
E_PER_W = N_EDGES // NW
ROWS_PER_TILE = N_NODES // NS


def _make_sc_aggregate(d: int, chunk: int, n_chunks: int, acc_rows: int):
  mesh = plsc.VectorSubcoreMesh(core_axis_name="c", subcore_axis_name="s")
  zrows = acc_rows // NS
  half = (n_chunks + 1) // 2

  @functools.partial(
      pl.kernel,
      out_type=jax.ShapeDtypeStruct((NC * N_NODES, d), jnp.float32),
      mesh=mesh,
      scratch_types=dict(
          src_h=pltpu.VMEM((half, chunk), jnp.int32),
          dst_h=pltpu.VMEM((half, chunk), jnp.int32),
          buf_a=pltpu.VMEM((chunk, d), jnp.float32),
          buf_b=pltpu.VMEM((chunk, d), jnp.float32),
          acc_sh=pltpu.VMEM_SHARED((acc_rows, d), jnp.float32),
          sem_a=pltpu.SemaphoreType.DMA,
          sem_b=pltpu.SemaphoreType.DMA,
      ),
      compiler_params=pltpu.CompilerParams(use_tc_tiling_on_sc=False),
  )
  def agg(table_hbm, ei_hbm, zeros_hbm, out_hbm,
          src_h, dst_h, buf_a, buf_b, acc_sh, sem_a, sem_b):
    c = lax.axis_index("c")
    s = lax.axis_index("s")
    wid = c * NS + s

    def gather(l, buf, sem):
      pltpu.async_copy(table_hbm.at[src_h.at[l]], buf, sem)

    def wait(l, buf, sem):
      pltpu.make_async_copy(table_hbm.at[src_h.at[l]], buf, sem).wait()

    def scatter(l, buf):
      pltpu.sync_copy(buf, acc_sh.at[dst_h.at[l]], add=True)

    def stage_idx(base, n):
      pltpu.sync_copy(ei_hbm.at[0, wid, pl.ds(base, n)], src_h.at[pl.ds(0, n)])
      pltpu.sync_copy(ei_hbm.at[1, wid, pl.ds(base, n)], dst_h.at[pl.ds(0, n)])

    def run_phase(n):
      gather(0, buf_a, sem_a)
      n_pairs = (n - 1) // 2

      def body(j, carry):
        l = 2 * j + 1
        gather(l, buf_b, sem_b)
        wait(l - 1, buf_a, sem_a)
        scatter(l - 1, buf_a)
        gather(l + 1, buf_a, sem_a)
        wait(l, buf_b, sem_b)
        scatter(l, buf_b)
        return carry

      lax.fori_loop(0, n_pairs, body, 0)
      if (n - 1) % 2 == 0:
        wait(n - 1, buf_a, sem_a)
        scatter(n - 1, buf_a)
      else:
        gather(n - 1, buf_b, sem_b)
        wait(n - 2, buf_a, sem_a)
        scatter(n - 2, buf_a)
        wait(n - 1, buf_b, sem_b)
        scatter(n - 1, buf_b)

    stage_idx(0, half)
    pltpu.sync_copy(zeros_hbm, buf_b)
    z0 = s * zrows
    zn_full, zrem = divmod(zrows, chunk)
    for k in range(zn_full):
      pltpu.sync_copy(buf_b, acc_sh.at[pl.ds(z0 + k * chunk, chunk)])
    if zrem:
      pltpu.sync_copy(buf_b.at[pl.ds(0, zrem)],
                      acc_sh.at[pl.ds(z0 + zn_full * chunk, zrem)])
    plsc.subcore_barrier()

    run_phase(half)
    stage_idx(half, n_chunks - half)
    run_phase(n_chunks - half)
    plsc.subcore_barrier()

    row0 = s * ROWS_PER_TILE
    out0 = c * N_NODES + row0
    n_full, rem = divmod(ROWS_PER_TILE, chunk)
    for k in range(n_full):
      pltpu.sync_copy(acc_sh.at[pl.ds(row0 + k * chunk, chunk)], buf_a)
      pltpu.sync_copy(buf_a, out_hbm.at[pl.ds(out0 + k * chunk, chunk)])
    if rem:
      pltpu.sync_copy(acc_sh.at[pl.ds(row0 + n_full * chunk, rem)],
                      buf_b.at[pl.ds(0, rem)])
      pltpu.sync_copy(buf_b.at[pl.ds(0, rem)],
                      out_hbm.at[pl.ds(out0 + n_full * chunk, rem)])

  return agg


_D1 = 144
_D2 = 48
_BLK = 1000
_GRID = N_NODES // _BLK


def _dense1_body(x_ref, a0_ref, a1_ref, ws1_ref, wn1_ref, b1_ref,
                 ws2_ref, wn2_ref, b2_ref, g2p_ref, s2b_ref, dinv_ref):
  a = a0_ref[...] + a1_ref[...]
  dinv = 1.0 / jnp.maximum(a[:, 128:129], 1.0)
  mean = a[:, :128] * dinv
  h1 = x_ref[...] @ ws1_ref[...] + mean @ wn1_ref[...] + b1_ref[...]
  h1 = jnp.maximum(h1, 0.0)
  g2 = h1 @ wn2_ref[...]
  g2p_ref[...] = jnp.concatenate(
      [g2, jnp.zeros((g2.shape[0], _D2 - g2.shape[1]), jnp.float32)], axis=1)
  s2b_ref[...] = h1 @ ws2_ref[...] + b2_ref[...]
  dinv_ref[...] = jnp.broadcast_to(dinv, (dinv.shape[0], 8))


def _dense2_body(a0_ref, a1_ref, s2b_ref, dinv_ref, out_ref):
  a = a0_ref[...] + a1_ref[...]
  out_ref[...] = s2b_ref[...] + a[:, :40] * dinv_ref[:, :1]


_C1, _NCH1 = 80, 125
_C2, _NCH2 = 80, 125


def kernel(x, edge_index, W_self1, W_neigh1, b1, W_self2, W_neigh2, b2):
  ei4 = edge_index.astype(jnp.int32).reshape(2, NW, _NCH1, _C1)
  n_classes = W_self2.shape[1]

  xp = jnp.concatenate(
      [x, jnp.ones((N_NODES, 1), jnp.float32),
       jnp.zeros((N_NODES, _D1 - x.shape[1] - 1), jnp.float32)], axis=1)
  z1 = jnp.zeros((_C1, _D1), jnp.float32)
  acc1 = _make_sc_aggregate(_D1, _C1, _NCH1, N_NODES)(xp, ei4, z1)

  g2p, s2b, dinv = pl.pallas_call(
      _dense1_body,
      grid=(_GRID,),
      in_specs=[
          pl.BlockSpec((_BLK, 128), lambda i: (i, 0)),
          pl.BlockSpec((_BLK, _D1), lambda i: (i, 0)),
          pl.BlockSpec((_BLK, _D1), lambda i: (i + _GRID, 0)),
          pl.BlockSpec((128, 128), lambda i: (0, 0)),
          pl.BlockSpec((128, 128), lambda i: (0, 0)),
          pl.BlockSpec((1, 128), lambda i: (0, 0)),
          pl.BlockSpec((128, n_classes), lambda i: (0, 0)),
          pl.BlockSpec((128, n_classes), lambda i: (0, 0)),
          pl.BlockSpec((1, n_classes), lambda i: (0, 0)),
      ],
      out_specs=[
          pl.BlockSpec((_BLK, _D2), lambda i: (i, 0)),
          pl.BlockSpec((_BLK, n_classes), lambda i: (i, 0)),
          pl.BlockSpec((_BLK, 8), lambda i: (i, 0)),
      ],
      out_shape=[
          jax.ShapeDtypeStruct((N_NODES, _D2), jnp.float32),
          jax.ShapeDtypeStruct((N_NODES, n_classes), jnp.float32),
          jax.ShapeDtypeStruct((N_NODES, 8), jnp.float32),
      ],
  )(x, acc1, acc1, W_self1, W_neigh1, b1.reshape(1, -1),
    W_self2, W_neigh2, b2.reshape(1, -1))

  z2 = jnp.zeros((_C2, _D2), jnp.float32)
  acc2 = _make_sc_aggregate(_D2, _C2, _NCH2, N_NODES)(g2p, ei4, z2)

  out = pl.pallas_call(
      _dense2_body,
      grid=(_GRID,),
      in_specs=[
          pl.BlockSpec((_BLK, _D2), lambda i: (i, 0)),
          pl.BlockSpec((_BLK, _D2), lambda i: (i + _GRID, 0)),
          pl.BlockSpec((_BLK, n_classes), lambda i: (i, 0)),
          pl.BlockSpec((_BLK, 8), lambda i: (i, 0)),
      ],
      out_specs=pl.BlockSpec((_BLK, n_classes), lambda i: (i, 0)),
      out_shape=jax.ShapeDtypeStruct((N_NODES, n_classes), jnp.float32),
  )(acc2, acc2, s2b, dinv)
  return out
